# baseline (speedup 1.0000x reference)
import functools

import jax
import jax.numpy as jnp
from jax import lax
from jax.experimental import pallas as pl
from jax.experimental.pallas import tpu as pltpu
from jax.experimental.pallas import tpu_sc as plsc

VOCAB = 100000
DIM = 32
B = 4096
CTX = 20

NC = 2
NS = 16
NW = NC * NS
BPW = B // NW

BT = 1024
VT = 4096
NB = B // BT
NV = (VOCAB + VT - 1) // VT
VPAD = NV * VT - VOCAB
NLANE = 128
NCH = VT // NLANE


def _tree(op, xs):
    while len(xs) > 1:
        nxt = [op(xs[i], xs[i + 1]) for i in range(0, len(xs) - 1, 2)]
        if len(xs) % 2:
            nxt.append(xs[-1])
        xs = nxt
    return xs[0]


def _sc_gather_body(inputs_t_hbm, target_hbm, emb_hbm, linw_hbm, linb2d_hbm,
                    sum_out, wt_out, bt_out,
                    idx_v, gath_v, sum_v, tgt_v, wt_v, bt_v, sem):
    wid = lax.axis_index("s") * NC + lax.axis_index("c")
    base = wid * BPW

    pltpu.sync_copy(inputs_t_hbm.at[:, pl.ds(base, BPW)], idx_v)

    copies = [
        pltpu.async_copy(emb_hbm.at[idx_v.at[j]], gath_v.at[j], sem)
        for j in range(CTX)
    ]
    for c in copies:
        c.wait()

    def row_body(r, _):
        acc0 = jnp.zeros((16,), jnp.float32)
        acc1 = jnp.zeros((16,), jnp.float32)
        for j in range(CTX):
            acc0 = acc0 + gath_v[j, r, pl.ds(0, 16)]
            acc1 = acc1 + gath_v[j, r, pl.ds(16, 16)]
        sum_v[r, pl.ds(0, 16)] = acc0
        sum_v[r, pl.ds(16, 16)] = acc1
        return 0

    lax.fori_loop(0, BPW, row_body, 0)
    pltpu.sync_copy(sum_v, sum_out.at[pl.ds(base, BPW)])

    pltpu.sync_copy(target_hbm.at[pl.ds(base, BPW)], tgt_v)
    pltpu.async_copy(linw_hbm.at[tgt_v], wt_v, sem).wait()
    pltpu.async_copy(linb2d_hbm.at[tgt_v], bt_v, sem).wait()
    pltpu.sync_copy(wt_v, wt_out.at[pl.ds(base, BPW)])
    pltpu.sync_copy(bt_v, bt_out.at[pl.ds(base, BPW)])


def _make_sc_gather():
    return pl.kernel(
        _sc_gather_body,
        out_type=(
            jax.ShapeDtypeStruct((B, DIM), jnp.float32),
            jax.ShapeDtypeStruct((B, DIM), jnp.float32),
            jax.ShapeDtypeStruct((B, 1), jnp.float32),
        ),
        mesh=plsc.VectorSubcoreMesh(core_axis_name="c", subcore_axis_name="s"),
        scratch_types=(
            pltpu.VMEM((CTX, BPW), jnp.int32),
            pltpu.VMEM((CTX, BPW, DIM), jnp.float32),
            pltpu.VMEM((BPW, DIM), jnp.float32),
            pltpu.VMEM((BPW,), jnp.int32),
            pltpu.VMEM((BPW, DIM), jnp.float32),
            pltpu.VMEM((BPW, 1), jnp.float32),
            pltpu.SemaphoreType.DMA,
        ),
        compiler_params=pltpu.CompilerParams(use_tc_tiling_on_sc=False),
    )


LOG2E = 1.4426950408889634
LN2 = 0.6931471805599453


def _tc_lse_body(se_ref, w_ref, wbt_ref, out_ref, m_ref, l_ref, acc_ref):
    bi = pl.program_id(0)
    vi = pl.program_id(1)

    @pl.when(vi == 0)
    def _():
        m_ref[...] = jnp.full((BT, NLANE), -jnp.inf, jnp.bfloat16)
        l_ref[...] = jnp.zeros((BT, NLANE), jnp.float32)

    se = se_ref[...]
    logits2 = lax.dot_general(
        se, w_ref[...], (((1,), (1,)), ((), ())),
        preferred_element_type=jnp.float32,
    ).astype(jnp.bfloat16)

    chunks = [logits2[:, j * NLANE:(j + 1) * NLANE] for j in range(NCH)]
    m_old = m_ref[...]
    m_new = jnp.maximum(m_old, _tree(jnp.maximum, chunks))
    s = _tree(lax.add, [jnp.exp2(c - m_new) for c in chunks]).astype(jnp.float32)
    scale = jnp.exp2((m_old - m_new).astype(jnp.float32))
    l_new = l_ref[...] * scale + s
    m_ref[...] = m_new
    l_ref[...] = l_new

    @pl.when(vi == NV - 1)
    def _():
        m32 = m_new.astype(jnp.float32)
        mx = jnp.max(m32, axis=1, keepdims=True)
        lx = jnp.sum(l_new * jnp.exp2(m32 - mx), axis=1, keepdims=True)
        tl2 = jnp.sum(se.astype(jnp.float32) * wbt_ref[...], axis=1,
                      keepdims=True)
        part = jnp.sum((mx - tl2) * LN2 + jnp.log(lx))
        prev = jnp.where(bi == 0, 0.0, acc_ref[0, 0])
        total = prev + part
        acc_ref[0, 0] = total

        @pl.when(bi == NB - 1)
        def _():
            out_ref[...] = jnp.broadcast_to(total / B, (1, 1))


_tc_lse = pl.pallas_call(
    _tc_lse_body,
    grid=(NB, NV),
    in_specs=[
        pl.BlockSpec((BT, DIM + 1), lambda b, v: (b, 0)),
        pl.BlockSpec((VT, DIM + 1), lambda b, v: (v, 0)),
        pl.BlockSpec((BT, DIM + 1), lambda b, v: (b, 0)),
    ],
    out_specs=pl.BlockSpec((1, 1), lambda b, v: (0, 0)),
    out_shape=jax.ShapeDtypeStruct((1, 1), jnp.float32),
    scratch_shapes=[
        pltpu.VMEM((BT, NLANE), jnp.bfloat16),
        pltpu.VMEM((BT, NLANE), jnp.float32),
        pltpu.SMEM((1, 1), jnp.float32),
    ],
)


@jax.jit
def kernel(inputs, target, emb_table, lin_w, lin_b):
    inputs_t = inputs.T.reshape(CTX, B)
    linb2d = lin_b.reshape(VOCAB, 1)

    sum_emb, wt, bt = _make_sc_gather()(inputs_t, target, emb_table, lin_w, linb2d)

    se1 = jnp.concatenate(
        [sum_emb, jnp.ones((B, 1), jnp.float32)], axis=1).astype(jnp.bfloat16)
    wb = jnp.concatenate([lin_w, lin_b[:, None]], axis=1) * LOG2E
    pad = jnp.concatenate(
        [jnp.zeros((VPAD, DIM), jnp.float32),
         jnp.full((VPAD, 1), -1e30, jnp.float32)], axis=1)
    wb_pad = jnp.concatenate([wb, pad], axis=0).astype(jnp.bfloat16)
    wbt = jnp.concatenate([wt, bt], axis=1) * LOG2E

    loss = _tc_lse(se1, wb_pad, wbt)
    return loss[0, 0]


# speedup vs baseline: 1.0399x; 1.0399x over previous
import functools

import jax
import jax.numpy as jnp
from jax import lax
from jax.experimental import pallas as pl
from jax.experimental.pallas import tpu as pltpu
from jax.experimental.pallas import tpu_sc as plsc

VOCAB = 100000
DIM = 32
B = 4096
CTX = 20

NC = 2
NS = 16
NW = NC * NS
BPW = B // NW

BT = 1024
VT = 4096
NB = B // BT
NV = (VOCAB + VT - 1) // VT
VPAD = NV * VT - VOCAB
NLANE = 128
NCH = VT // NLANE


def _tree(op, xs):
    while len(xs) > 1:
        nxt = [op(xs[i], xs[i + 1]) for i in range(0, len(xs) - 1, 2)]
        if len(xs) % 2:
            nxt.append(xs[-1])
        xs = nxt
    return xs[0]


def _sc_gather_body(inputs_t_hbm, target_hbm, emb_hbm, linw_hbm, linb2d_hbm,
                    sum_out, wt_out, bt_out,
                    idx_v, gath_v, sum_v, tgt_v, wt_v, bt_v, sem):
    wid = lax.axis_index("s") * NC + lax.axis_index("c")
    base = wid * BPW

    pltpu.sync_copy(inputs_t_hbm.at[:, pl.ds(base, BPW)], idx_v)

    copies = [
        pltpu.async_copy(emb_hbm.at[idx_v.at[j]], gath_v.at[j], sem)
        for j in range(CTX)
    ]
    for c in copies:
        c.wait()

    def row_body(r, _):
        acc0 = jnp.zeros((16,), jnp.float32)
        acc1 = jnp.zeros((16,), jnp.float32)
        for j in range(CTX):
            acc0 = acc0 + gath_v[j, r, pl.ds(0, 16)]
            acc1 = acc1 + gath_v[j, r, pl.ds(16, 16)]
        sum_v[r, pl.ds(0, 16)] = acc0
        sum_v[r, pl.ds(16, 16)] = acc1
        return 0

    lax.fori_loop(0, BPW, row_body, 0)
    pltpu.sync_copy(sum_v, sum_out.at[pl.ds(base, BPW)])

    pltpu.sync_copy(target_hbm.at[pl.ds(base, BPW)], tgt_v)
    pltpu.async_copy(linw_hbm.at[tgt_v], wt_v, sem).wait()
    pltpu.async_copy(linb2d_hbm.at[tgt_v], bt_v, sem).wait()
    pltpu.sync_copy(wt_v, wt_out.at[pl.ds(base, BPW)])
    pltpu.sync_copy(bt_v, bt_out.at[pl.ds(base, BPW)])


def _make_sc_gather():
    return pl.kernel(
        _sc_gather_body,
        out_type=(
            jax.ShapeDtypeStruct((B, DIM), jnp.float32),
            jax.ShapeDtypeStruct((B, DIM), jnp.float32),
            jax.ShapeDtypeStruct((B, 1), jnp.float32),
        ),
        mesh=plsc.VectorSubcoreMesh(core_axis_name="c", subcore_axis_name="s"),
        scratch_types=(
            pltpu.VMEM((CTX, BPW), jnp.int32),
            pltpu.VMEM((CTX, BPW, DIM), jnp.float32),
            pltpu.VMEM((BPW, DIM), jnp.float32),
            pltpu.VMEM((BPW,), jnp.int32),
            pltpu.VMEM((BPW, DIM), jnp.float32),
            pltpu.VMEM((BPW, 1), jnp.float32),
            pltpu.SemaphoreType.DMA,
        ),
        compiler_params=pltpu.CompilerParams(use_tc_tiling_on_sc=False),
    )


LOG2E = 1.4426950408889634
LN2 = 0.6931471805599453


def _tc_lse_body(se_ref, w_ref, wbt_ref, out_ref, m_ref, l_ref, acc_ref):
    bi = pl.program_id(0)
    vi = pl.program_id(1)

    @pl.when(vi == 0)
    def _():
        m_ref[...] = jnp.full((BT, NLANE), -jnp.inf, jnp.bfloat16)
        l_ref[...] = jnp.zeros((BT, NLANE), jnp.float32)

    se = se_ref[...]
    logits2 = lax.dot_general(
        se, w_ref[...], (((1,), (0,)), ((), ())),
        preferred_element_type=jnp.float32,
    ).astype(jnp.bfloat16)

    m_old = m_ref[...]
    chunks = []
    macc = m_old
    for j in range(NCH):
        c = logits2[:, j * NLANE:(j + 1) * NLANE]
        chunks.append(c)
        macc = jnp.maximum(macc, c)
    m_new = macc
    s = _tree(lax.add, [jnp.exp2(c - m_new) for c in chunks]).astype(jnp.float32)
    scale = jnp.exp2((m_old - m_new).astype(jnp.float32))
    l_new = l_ref[...] * scale + s
    m_ref[...] = m_new
    l_ref[...] = l_new

    @pl.when(vi == NV - 1)
    def _():
        m32 = m_new.astype(jnp.float32)
        mx = jnp.max(m32, axis=1, keepdims=True)
        lx = jnp.sum(l_new * jnp.exp2(m32 - mx), axis=1, keepdims=True)
        tl2 = jnp.sum(se.astype(jnp.float32) * wbt_ref[...], axis=1,
                      keepdims=True)
        part = jnp.sum((mx - tl2) * LN2 + jnp.log(lx))
        prev = jnp.where(bi == 0, 0.0, acc_ref[0, 0])
        total = prev + part
        acc_ref[0, 0] = total

        @pl.when(bi == NB - 1)
        def _():
            out_ref[...] = jnp.broadcast_to(total / B, (1, 1))


_tc_lse = pl.pallas_call(
    _tc_lse_body,
    grid=(NB, NV),
    in_specs=[
        pl.BlockSpec((BT, DIM + 1), lambda b, v: (b, 0)),
        pl.BlockSpec((DIM + 1, VT), lambda b, v: (0, v)),
        pl.BlockSpec((BT, DIM + 1), lambda b, v: (b, 0)),
    ],
    out_specs=pl.BlockSpec((1, 1), lambda b, v: (0, 0)),
    out_shape=jax.ShapeDtypeStruct((1, 1), jnp.float32),
    scratch_shapes=[
        pltpu.VMEM((BT, NLANE), jnp.bfloat16),
        pltpu.VMEM((BT, NLANE), jnp.float32),
        pltpu.SMEM((1, 1), jnp.float32),
    ],
)


@jax.jit
def kernel(inputs, target, emb_table, lin_w, lin_b):
    inputs_t = inputs.T.reshape(CTX, B)
    linb2d = lin_b.reshape(VOCAB, 1)

    sum_emb, wt, bt = _make_sc_gather()(inputs_t, target, emb_table, lin_w, linb2d)

    se1 = jnp.concatenate(
        [sum_emb, jnp.ones((B, 1), jnp.float32)], axis=1).astype(jnp.bfloat16)
    wb = jnp.concatenate([lin_w, lin_b[:, None]], axis=1) * LOG2E
    pad = jnp.concatenate(
        [jnp.zeros((VPAD, DIM), jnp.float32),
         jnp.full((VPAD, 1), -1e30, jnp.float32)], axis=1)
    wb_pad = jnp.concatenate([wb, pad], axis=0).astype(jnp.bfloat16).T
    wbt = jnp.concatenate([wt, bt], axis=1) * LOG2E

    loss = _tc_lse(se1, wb_pad, wbt)
    return loss[0, 0]
